# Initial kernel scaffold; baseline (speedup 1.0000x reference)
import jax
import jax.numpy as jnp
from jax.experimental import pallas as pl


def kernel(indices, embeddings):
    raise NotImplementedError("write your pallas kernel here")


# speedup vs baseline: 2.6965x; 2.6965x over previous
import functools

import jax
import jax.numpy as jnp
from jax import lax
from jax.experimental import pallas as pl
from jax.experimental.pallas import tpu as pltpu
from jax.experimental.pallas import tpu_sc as plsc

B = 16384
S = 50
D = 32
V = 1000000
L = 16
NC, NS = 2, 16
NW = NC * NS
ROWS_PER_W = B // NW
R = 32
CHUNKS = ROWS_PER_W // R
IDX_PER_CHUNK = R * S
GB = 100
NGATHER = IDX_PER_CHUNK // GB


def _make_sc_call():
  mesh = plsc.VectorSubcoreMesh(core_axis_name="c", subcore_axis_name="s")

  @functools.partial(
      pl.kernel,
      out_type=jax.ShapeDtypeStruct((B, D), jnp.float32),
      mesh=mesh,
      compiler_params=pltpu.CompilerParams(needs_layout_passes=False,
                                           use_tc_tiling_on_sc=False),
      scratch_types=[
          pltpu.VMEM((NGATHER, GB), jnp.int32),
          pltpu.VMEM((IDX_PER_CHUNK,), jnp.int32),
          pltpu.VMEM((IDX_PER_CHUNK, D), jnp.float32),
          pltpu.VMEM((R, D), jnp.float32),
          pltpu.VMEM((L,), jnp.float32),
          pltpu.VMEM((L,), jnp.float32),
          pltpu.VMEM((1, D), jnp.float32),
          pltpu.SemaphoreType.DMA,
      ],
  )
  def sc_kernel(idx2_hbm, idxf_hbm, emb_hbm, out_hbm,
                idx2_v, idxf_v, rows_v, out_v, inv_v, nz_v, e0_v, sem):
    wid = lax.axis_index("s") * NC + lax.axis_index("c")

    pltpu.sync_copy(emb_hbm.at[pl.ds(0, 1)], e0_v)
    e00 = e0_v[0, 0:L]
    e01 = e0_v[0, L:D]
    lanes = lax.iota(jnp.int32, L)
    lanes_s = lanes * S

    def chunk_body(c, carry):
      row0 = pl.multiple_of(wid * ROWS_PER_W + c * R, R)
      pltpu.sync_copy(
          idx2_hbm.at[pl.ds(pl.multiple_of(row0 * S // GB, NGATHER), NGATHER)],
          idx2_v)
      pltpu.sync_copy(
          idxf_hbm.at[pl.ds(pl.multiple_of(row0 * S, IDX_PER_CHUNK),
                            IDX_PER_CHUNK)], idxf_v)
      descs = []
      for j in range(NGATHER):
        descs.append(pltpu.async_copy(
            emb_hbm.at[idx2_v.at[j]],
            rows_v.at[pl.ds(j * GB, GB)], sem))
      for dd in descs:
        dd.wait()

      for g in range(R // L):
        gbase = g * L * S
        cnt = jnp.zeros((L,), jnp.int32)
        for s in range(S):
          vals = plsc.load_gather(idxf_v, [lanes_s + (gbase + s)])
          cnt = cnt + (vals != 0).astype(jnp.int32)
        cntf = cnt.astype(jnp.float32)
        inv_v[...] = jnp.where(cnt == 0, 0.0, 1.0 / (cntf + 1e-8))
        nz_v[...] = jnp.float32(S) - cntf

        def row_body(i, _, g=g):
          lr = g * L + i
          fbase = lr * S
          acc0 = rows_v[fbase, 0:L]
          acc1 = rows_v[fbase, L:D]
          for s in range(1, S):
            acc0 = acc0 + rows_v[fbase + s, 0:L]
            acc1 = acc1 + rows_v[fbase + s, L:D]
          isplat = jnp.full((L,), i, jnp.int32)
          nz = plsc.load_gather(nz_v, [isplat])
          inv = plsc.load_gather(inv_v, [isplat])
          out_v[lr, 0:L] = (acc0 - nz * e00) * inv
          out_v[lr, L:D] = (acc1 - nz * e01) * inv
          return 0

        lax.fori_loop(0, L, row_body, 0)

      pltpu.sync_copy(out_v, out_hbm.at[pl.ds(pl.multiple_of(row0, R), R)])
      return carry

    lax.fori_loop(0, CHUNKS, chunk_body, 0)

  return sc_kernel


_make_sc_call = functools.cache(_make_sc_call)


def kernel(indices, embeddings):
  idx = indices.astype(jnp.int32)
  idx2 = idx.reshape(B * S // GB, GB)
  idxf = idx.reshape(B * S)
  return _make_sc_call()(idx2, idxf, embeddings)


# speedup vs baseline: 2.9956x; 1.1109x over previous
import functools

import jax
import jax.numpy as jnp
from jax import lax
from jax.experimental import pallas as pl
from jax.experimental.pallas import tpu as pltpu
from jax.experimental.pallas import tpu_sc as plsc

B = 16384
S = 50
D = 32
L = 16
NC, NS = 2, 16
NW = NC * NS
RW = B // NW
GB = 128
KB = RW // GB
CB = B // GB


def _make_sc_call():
  mesh = plsc.VectorSubcoreMesh(core_axis_name="c", subcore_axis_name="s")

  @functools.partial(
      pl.kernel,
      out_type=jax.ShapeDtypeStruct((B, D), jnp.float32),
      mesh=mesh,
      compiler_params=pltpu.CompilerParams(needs_layout_passes=False,
                                           use_tc_tiling_on_sc=False),
      scratch_types=[
          pltpu.VMEM((S, KB, GB), jnp.int32),
          pltpu.VMEM((RW, D), jnp.float32),
          pltpu.VMEM((RW,), jnp.float32),
          pltpu.VMEM((RW,), jnp.float32),
          pltpu.VMEM((1, D), jnp.float32),
          pltpu.SemaphoreType.DMA,
      ],
  )
  def sc_kernel(idxt_hbm, emb_hbm, out_hbm,
                idxt_v, acc_v, inv_v, nz_v, e0_v, sem):
    wid = lax.axis_index("s") * NC + lax.axis_index("c")
    cb0 = pl.multiple_of(wid * KB, KB)
    row0 = pl.multiple_of(wid * RW, RW)

    pltpu.sync_copy(emb_hbm.at[pl.ds(0, 1)], e0_v)
    pltpu.sync_copy(idxt_hbm.at[:, pl.ds(cb0, KB)], idxt_v)

    d0 = [pltpu.async_copy(emb_hbm.at[idxt_v.at[0, k]],
                           acc_v.at[pl.ds(k * GB, GB)], sem)
          for k in range(KB)]
    for dd in d0:
      dd.wait()
    descs = [pltpu.async_copy(emb_hbm.at[idxt_v.at[s, k]],
                              acc_v.at[pl.ds(k * GB, GB)], sem, add=True)
             for s in range(1, S) for k in range(KB)]

    e00 = e0_v[0, 0:L]
    e01 = e0_v[0, L:D]
    for k in range(KB):
      for l in range(GB // L):
        off = l * L

        def cnt_body(s, cnt, k=k, off=off):
          return cnt + (idxt_v[s, k, pl.ds(off, L)] != 0).astype(jnp.int32)

        cnt = lax.fori_loop(0, S, cnt_body, jnp.zeros((L,), jnp.int32))
        cntf = cnt.astype(jnp.float32)
        gbase = k * GB + off
        inv_v[pl.ds(gbase, L)] = jnp.where(cnt == 0, 0.0,
                                           1.0 / (cntf + 1e-8))
        nz_v[pl.ds(gbase, L)] = jnp.float32(S) - cntf

    for dd in descs:
      dd.wait()

    def row_body(r, carry):
      isplat = jnp.full((L,), r, jnp.int32)
      nz = plsc.load_gather(nz_v, [isplat])
      inv = plsc.load_gather(inv_v, [isplat])
      acc_v[r, 0:L] = (acc_v[r, 0:L] - nz * e00) * inv
      acc_v[r, L:D] = (acc_v[r, L:D] - nz * e01) * inv
      return carry

    lax.fori_loop(0, RW, row_body, 0)

    pltpu.sync_copy(acc_v, out_hbm.at[pl.ds(row0, RW)])

  return sc_kernel


_make_sc_call = functools.cache(_make_sc_call)


def kernel(indices, embeddings):
  idxt = indices.astype(jnp.int32).T.reshape(S, CB, GB)
  return _make_sc_call()(idxt, embeddings)
